# baseline (speedup 1.0000x reference)
import jax
import jax.numpy as jnp
from jax import lax
from jax.experimental import pallas as pl
from jax.experimental.pallas import tpu as pltpu
from jax.experimental.pallas import tpu_sc as plsc

T_TEXT = 200
EMBED_DIM = 1024
ROWS_PER_WORKER = 8
NUM_ACTIVE = T_TEXT // ROWS_PER_WORKER
NUM_CORES = 2


def _gather_body(idx_hbm, table_hbm, out_hbm, idx_v, rows_v, sem):
    wid = lax.axis_index("s") * NUM_CORES + lax.axis_index("c")

    @pl.when(wid < NUM_ACTIVE)
    def _():
        base = wid * ROWS_PER_WORKER
        pltpu.sync_copy(idx_hbm.at[pl.ds(base, ROWS_PER_WORKER)], idx_v)
        pltpu.async_copy(table_hbm.at[idx_v], rows_v, sem).wait()
        pltpu.sync_copy(rows_v, out_hbm.at[pl.ds(base, ROWS_PER_WORKER)])


def kernel(token_ids, embed_table):
    idx = token_ids.reshape(T_TEXT).astype(jnp.int32)
    mesh = plsc.VectorSubcoreMesh(core_axis_name="c", subcore_axis_name="s")
    out = pl.kernel(
        _gather_body,
        mesh=mesh,
        out_type=jax.ShapeDtypeStruct((T_TEXT, EMBED_DIM), jnp.float32),
        scratch_types=[
            pltpu.VMEM((ROWS_PER_WORKER,), jnp.int32),
            pltpu.VMEM((ROWS_PER_WORKER, EMBED_DIM), jnp.float32),
            pltpu.SemaphoreType.DMA,
        ],
    )(idx, embed_table)
    return out.reshape(1, T_TEXT, EMBED_DIM)


# speedup vs baseline: 1.0417x; 1.0417x over previous
import jax
import jax.numpy as jnp
from jax import lax
from jax.experimental import pallas as pl
from jax.experimental.pallas import tpu as pltpu
from jax.experimental.pallas import tpu_sc as plsc

T_TEXT = 200
EMBED_DIM = 1024
ROWS_MAIN = 16
NUM_MAIN = 12
ROWS_TAIL = 8


def _gather_body(idx_hbm, table_hbm, out_hbm, idx_v, rows_v, sem):
    wid = lax.axis_index("s")

    @pl.when(wid < NUM_MAIN)
    def _():
        base = wid * ROWS_MAIN
        pltpu.sync_copy(idx_hbm.at[pl.ds(base, ROWS_MAIN)], idx_v)
        pltpu.async_copy(table_hbm.at[idx_v], rows_v, sem).wait()
        pltpu.sync_copy(rows_v, out_hbm.at[pl.ds(base, ROWS_MAIN)])

    @pl.when(wid == NUM_MAIN)
    def _():
        base = NUM_MAIN * ROWS_MAIN
        pltpu.sync_copy(
            idx_hbm.at[pl.ds(base, ROWS_TAIL)], idx_v.at[pl.ds(0, ROWS_TAIL)]
        )
        pltpu.async_copy(
            table_hbm.at[idx_v.at[pl.ds(0, ROWS_TAIL)]],
            rows_v.at[pl.ds(0, ROWS_TAIL)],
            sem,
        ).wait()
        pltpu.sync_copy(
            rows_v.at[pl.ds(0, ROWS_TAIL)], out_hbm.at[pl.ds(base, ROWS_TAIL)]
        )


def kernel(token_ids, embed_table):
    idx = token_ids.reshape(T_TEXT).astype(jnp.int32)
    mesh = plsc.VectorSubcoreMesh(
        core_axis_name="c", subcore_axis_name="s", num_cores=1
    )
    out = pl.kernel(
        _gather_body,
        mesh=mesh,
        out_type=jax.ShapeDtypeStruct((T_TEXT, EMBED_DIM), jnp.float32),
        scratch_types=[
            pltpu.VMEM((ROWS_MAIN,), jnp.int32),
            pltpu.VMEM((ROWS_MAIN, EMBED_DIM), jnp.float32),
            pltpu.SemaphoreType.DMA,
        ],
    )(idx, embed_table)
    return out.reshape(1, T_TEXT, EMBED_DIM)
